# baseline (speedup 1.0000x reference)
import functools

import jax
import jax.numpy as jnp
from jax import lax
from jax.experimental import pallas as pl
from jax.experimental.pallas import tpu as pltpu
from jax.experimental.pallas import tpu_sc as plsc

N_RANK = 64
BATCH = 16384

NUM_CORES = 2
NUM_SUBCORES = 16
NUM_WORKERS = NUM_CORES * NUM_SUBCORES
N_PER_W = BATCH // NUM_WORKERS
LANES = 16
N_GROUPS = N_PER_W // LANES
ROW_W = 2 * N_RANK
CHUNK = 4

SELU_ALPHA = 1.6732632423543772
SELU_SCALE = 1.0507009873554805

_mesh = plsc.VectorSubcoreMesh(core_axis_name="c", subcore_axis_name="s")


@functools.partial(
    pl.kernel,
    out_type=jax.ShapeDtypeStruct((BATCH,), jnp.float32),
    mesh=_mesh,
    compiler_params=pltpu.CompilerParams(
        needs_layout_passes=False, use_tc_tiling_on_sc=True),
    scratch_types=[
        pltpu.VMEM((N_PER_W,), jnp.int32),
        pltpu.VMEM((N_PER_W,), jnp.int32),
        pltpu.VMEM((N_PER_W, ROW_W), jnp.float32),
        pltpu.VMEM((CHUNK, N_RANK, 128), jnp.float32),
        pltpu.VMEM((N_PER_W,), jnp.float32),
        pltpu.VMEM((N_PER_W,), jnp.float32),
        pltpu.VMEM((LANES,), jnp.float32),
        pltpu.VMEM((N_PER_W,), jnp.float32),
        pltpu.SemaphoreType.DMA,
        pltpu.SemaphoreType.DMA,
        pltpu.SemaphoreType.DMA,
    ],
)
def _lfm_sc(uid_hbm, iid_hbm, pt_hbm, q_hbm, mu_hbm, ub_hbm, ib_hbm,
            out_hbm, uid_v, iid_v, rows_v, blocks_v, ub_v, ib_v, mu_v,
            out_v, sem, qsem, bsem):
    wid = lax.axis_index("s") * NUM_CORES + lax.axis_index("c")
    base = wid * N_PER_W

    pltpu.sync_copy(uid_hbm.at[pl.ds(base, N_PER_W)], uid_v)
    pltpu.sync_copy(iid_hbm.at[pl.ds(base, N_PER_W)], iid_v)
    pltpu.sync_copy(mu_hbm, mu_v)

    cb1 = pltpu.async_copy(ub_hbm.at[uid_v], ub_v, bsem)
    cb2 = pltpu.async_copy(ib_hbm.at[iid_v], ib_v, bsem)

    @pl.loop(0, N_GROUPS)
    def _(g):
        iv = iid_v[pl.ds(g * LANES, LANES)]
        for l in range(LANES):
            j = g * LANES + l
            pltpu.async_copy(q_hbm.at[iv[l]],
                             rows_v.at[j, pl.ds(N_RANK, N_RANK)], qsem)

    d16 = lax.iota(jnp.int32, LANES)

    @pl.loop(0, N_GROUPS)
    def _(g):
        uv = uid_v[pl.ds(g * LANES, LANES)]
        for sub in range(LANES // CHUNK):
            for l in range(CHUNK):
                s = uv[sub * CHUNK + l]
                cstart = pl.multiple_of((s // 128) * 128, 128)
                for k in range(N_RANK // 8):
                    pltpu.async_copy(
                        pt_hbm.at[k, :, pl.ds(cstart, 128)],
                        blocks_v.at[l, pl.ds(k * 8, 8), :], sem)
            for l in range(CHUNK):
                for k in range(N_RANK // 8):
                    pltpu.make_async_copy(
                        pt_hbm.at[0, :, pl.ds(0, 128)],
                        blocks_v.at[l, pl.ds(k * 8, 8), :], sem).wait()
            for l in range(CHUNK):
                j = g * LANES + sub * CHUNK + l
                s = uv[sub * CHUNK + l]
                lane16 = jnp.broadcast_to(s % 128, (LANES,))
                slot16 = jnp.full((LANES,), l, jnp.int32)
                j16 = jnp.broadcast_to(j, (LANES,))
                for k in range(N_RANK // LANES):
                    p16 = plsc.load_gather(
                        blocks_v, [slot16, k * LANES + d16, lane16])
                    plsc.store_scatter(rows_v, [j16, k * LANES + d16], p16)

    @pl.loop(0, N_PER_W)
    def _(j):
        pltpu.make_async_copy(q_hbm.at[0],
                              rows_v.at[j, pl.ds(N_RANK, N_RANK)],
                              qsem).wait()
    cb1.wait()
    cb2.wait()

    mu = mu_v[...]
    lane = lax.iota(jnp.int32, LANES)

    @pl.loop(0, N_GROUPS)
    def _(g):
        rows = g * LANES + lane
        acc = jnp.zeros((LANES,), jnp.float32)
        for d in range(N_RANK):
            u = plsc.load_gather(rows_v, [rows, jnp.full((LANES,), d, jnp.int32)])
            t = plsc.load_gather(rows_v, [rows, jnp.full((LANES,), N_RANK + d, jnp.int32)])
            x = u * t
            acc = acc + jnp.where(x > 0.0, x, SELU_ALPHA * (jnp.exp(x) - 1.0))
        sl = pl.ds(g * LANES, LANES)
        out_v[sl] = SELU_SCALE * acc + ub_v[sl] + ib_v[sl] + mu

    pltpu.sync_copy(out_v, out_hbm.at[pl.ds(base, N_PER_W)])


def kernel(user_ids, item_ids, P, Q, mu, user_bias, item_bias):
    uid = user_ids.astype(jnp.int32)
    iid = item_ids.astype(jnp.int32)
    mu16 = jnp.broadcast_to(mu.astype(jnp.float32), (LANES,))
    pt3 = P.T.reshape(8, 8, P.shape[0])
    return _lfm_sc(uid, iid, pt3, Q, mu16, user_bias, item_bias)


# speedup vs baseline: 1.1475x; 1.1475x over previous
import functools

import jax
import jax.numpy as jnp
from jax import lax
from jax.experimental import pallas as pl
from jax.experimental.pallas import tpu as pltpu
from jax.experimental.pallas import tpu_sc as plsc

N_RANK = 64
BATCH = 16384

NUM_CORES = 2
NUM_SUBCORES = 16
NUM_WORKERS = NUM_CORES * NUM_SUBCORES
N_PER_W = BATCH // NUM_WORKERS
LANES = 16
N_GROUPS = N_PER_W // LANES
ROW_W = 2 * N_RANK
CHUNK = 4

SELU_ALPHA = 1.6732632423543772
SELU_SCALE = 1.0507009873554805

_mesh = plsc.VectorSubcoreMesh(core_axis_name="c", subcore_axis_name="s")


@functools.partial(
    pl.kernel,
    out_type=jax.ShapeDtypeStruct((BATCH,), jnp.float32),
    mesh=_mesh,
    compiler_params=pltpu.CompilerParams(
        needs_layout_passes=False, use_tc_tiling_on_sc=True),
    scratch_types=[
        pltpu.VMEM((N_PER_W,), jnp.int32),
        pltpu.VMEM((N_PER_W,), jnp.int32),
        pltpu.VMEM((N_PER_W,), jnp.int32),
        pltpu.VMEM((N_PER_W, ROW_W), jnp.float32),
        pltpu.VMEM((CHUNK, N_RANK, 128), jnp.float32),
        pltpu.VMEM((N_PER_W,), jnp.float32),
        pltpu.VMEM((N_PER_W,), jnp.float32),
        pltpu.VMEM((LANES,), jnp.float32),
        pltpu.VMEM((N_PER_W,), jnp.float32),
        pltpu.SemaphoreType.DMA,
        pltpu.SemaphoreType.DMA,
        pltpu.SemaphoreType.DMA,
    ],
)
def _lfm_sc(uid_hbm, iid_hbm, ord_hbm, pt_hbm, q_hbm, mu_hbm, ub_hbm,
            ib_hbm, out_hbm, ord_v, uid_v, iid_v, rows_v, blocks_v,
            ub_v, ib_v, mu_v, out_v, sem, qsem, bsem):
    wid = lax.axis_index("s") * NUM_CORES + lax.axis_index("c")
    base = wid * N_PER_W

    pltpu.sync_copy(ord_hbm.at[pl.ds(base, N_PER_W)], ord_v)
    pltpu.sync_copy(mu_hbm, mu_v)
    c1 = pltpu.async_copy(uid_hbm.at[ord_v], uid_v, bsem)
    c2 = pltpu.async_copy(iid_hbm.at[ord_v], iid_v, bsem)
    c1.wait()
    c2.wait()
    cb1 = pltpu.async_copy(ub_hbm.at[uid_v], ub_v, bsem)
    cb2 = pltpu.async_copy(ib_hbm.at[iid_v], ib_v, bsem)

    @pl.loop(0, N_GROUPS)
    def _(g):
        iv = iid_v[pl.ds(g * LANES, LANES)]
        for l in range(LANES):
            j = g * LANES + l
            pltpu.async_copy(q_hbm.at[iv[l]],
                             rows_v.at[j, pl.ds(N_RANK, N_RANK)], qsem)

    d16 = lax.iota(jnp.int32, LANES)

    @pl.loop(0, N_GROUPS)
    def _(g):
        uv = uid_v[pl.ds(g * LANES, LANES)]
        for sub in range(LANES // CHUNK):
            ss = [uv[sub * CHUNK + l] for l in range(CHUNK)]
            cs = [s // 128 for s in ss]
            fired = [None] * CHUNK
            slots = [None] * CHUNK
            prev_slot = jnp.int32(0)
            for l in range(CHUNK):
                if l == 0:
                    fired[l] = None
                    slots[l] = jnp.int32(0)
                else:
                    f = cs[l] != cs[l - 1]
                    fired[l] = f
                    slots[l] = jnp.where(f, jnp.int32(l), slots[l - 1])
            for l in range(CHUNK):
                cstart = pl.multiple_of(cs[l] * 128, 128)

                def fire(l=l, cstart=cstart):
                    for k in range(N_RANK // 8):
                        pltpu.async_copy(
                            pt_hbm.at[k, :, pl.ds(cstart, 128)],
                            blocks_v.at[l, pl.ds(k * 8, 8), :], sem)

                if fired[l] is None:
                    fire()
                else:
                    pl.when(fired[l])(fire)
            for l in range(CHUNK):
                def drain(l=l):
                    for k in range(N_RANK // 8):
                        pltpu.make_async_copy(
                            pt_hbm.at[0, :, pl.ds(0, 128)],
                            blocks_v.at[l, pl.ds(k * 8, 8), :], sem).wait()

                if fired[l] is None:
                    drain()
                else:
                    pl.when(fired[l])(drain)
            for l in range(CHUNK):
                j = g * LANES + sub * CHUNK + l
                s = ss[l]
                lane16 = jnp.broadcast_to(s % 128, (LANES,))
                slot16 = jnp.broadcast_to(slots[l], (LANES,))
                j16 = jnp.broadcast_to(j, (LANES,))
                for k in range(N_RANK // LANES):
                    p16 = plsc.load_gather(
                        blocks_v, [slot16, k * LANES + d16, lane16])
                    plsc.store_scatter(rows_v, [j16, k * LANES + d16], p16)

    @pl.loop(0, N_PER_W)
    def _(j):
        pltpu.make_async_copy(q_hbm.at[0],
                              rows_v.at[j, pl.ds(N_RANK, N_RANK)],
                              qsem).wait()
    cb1.wait()
    cb2.wait()

    mu = mu_v[...]
    lane = lax.iota(jnp.int32, LANES)

    @pl.loop(0, N_GROUPS)
    def _(g):
        rows = g * LANES + lane
        acc = jnp.zeros((LANES,), jnp.float32)
        for d in range(N_RANK):
            u = plsc.load_gather(rows_v, [rows, jnp.full((LANES,), d, jnp.int32)])
            t = plsc.load_gather(rows_v, [rows, jnp.full((LANES,), N_RANK + d, jnp.int32)])
            x = u * t
            acc = acc + jnp.where(x > 0.0, x, SELU_ALPHA * (jnp.exp(x) - 1.0))
        sl = pl.ds(g * LANES, LANES)
        out_v[sl] = SELU_SCALE * acc + ub_v[sl] + ib_v[sl] + mu

    pltpu.async_copy(out_v, out_hbm.at[ord_v], qsem).wait()


def kernel(user_ids, item_ids, P, Q, mu, user_bias, item_bias):
    uid = user_ids.astype(jnp.int32)
    iid = item_ids.astype(jnp.int32)
    order = jnp.argsort(uid).astype(jnp.int32)
    mu16 = jnp.broadcast_to(mu.astype(jnp.float32), (LANES,))
    pt3 = P.T.reshape(8, 8, P.shape[0])
    return _lfm_sc(uid, iid, order, pt3, Q, mu16, user_bias, item_bias)


# speedup vs baseline: 1.3358x; 1.1641x over previous
import functools

import jax
import jax.numpy as jnp
from jax import lax
from jax.experimental import pallas as pl
from jax.experimental.pallas import tpu as pltpu
from jax.experimental.pallas import tpu_sc as plsc

N_RANK = 64
BATCH = 16384

NUM_CORES = 2
NUM_SUBCORES = 16
NUM_WORKERS = NUM_CORES * NUM_SUBCORES
N_PER_W = BATCH // NUM_WORKERS
LANES = 16
N_GROUPS = N_PER_W // LANES
ROW_W = 2 * N_RANK
CHUNK = 8
N_PASS = 2
N_PER_PASS = N_PER_W // N_PASS
G_PER_PASS = N_PER_PASS // LANES

SELU_ALPHA = 1.6732632423543772
SELU_SCALE = 1.0507009873554805

_mesh = plsc.VectorSubcoreMesh(core_axis_name="c", subcore_axis_name="s")


@functools.partial(
    pl.kernel,
    out_type=jax.ShapeDtypeStruct((BATCH,), jnp.float32),
    mesh=_mesh,
    compiler_params=pltpu.CompilerParams(
        needs_layout_passes=False, use_tc_tiling_on_sc=True),
    scratch_types=[
        pltpu.VMEM((N_PER_W,), jnp.int32),
        pltpu.VMEM((N_PER_W,), jnp.int32),
        pltpu.VMEM((N_PER_W,), jnp.int32),
        pltpu.VMEM((N_PER_PASS, ROW_W), jnp.float32),
        pltpu.VMEM((CHUNK, N_RANK, 128), jnp.float32),
        pltpu.VMEM((N_PER_W,), jnp.float32),
        pltpu.VMEM((N_PER_W,), jnp.float32),
        pltpu.VMEM((LANES,), jnp.float32),
        pltpu.VMEM((N_PER_W,), jnp.float32),
        pltpu.SemaphoreType.DMA,
        pltpu.SemaphoreType.DMA,
        pltpu.SemaphoreType.DMA,
    ],
)
def _lfm_sc(uid_hbm, iid_hbm, ord_hbm, pt_hbm, q_hbm, mu_hbm, ub_hbm,
            ib_hbm, out_hbm, ord_v, uid_v, iid_v, rows_v, blocks_v,
            ub_v, ib_v, mu_v, out_v, sem, qsem, bsem):
    wid = lax.axis_index("s") * NUM_CORES + lax.axis_index("c")
    base = wid * N_PER_W

    pltpu.sync_copy(ord_hbm.at[pl.ds(base, N_PER_W)], ord_v)
    pltpu.sync_copy(mu_hbm, mu_v)
    c1 = pltpu.async_copy(uid_hbm.at[ord_v], uid_v, bsem)
    c2 = pltpu.async_copy(iid_hbm.at[ord_v], iid_v, bsem)
    c1.wait()
    c2.wait()
    cb1 = pltpu.async_copy(ub_hbm.at[uid_v], ub_v, bsem)
    cb2 = pltpu.async_copy(ib_hbm.at[iid_v], ib_v, bsem)

    cb1.wait()
    cb2.wait()

    d16 = lax.iota(jnp.int32, LANES)
    mu = mu_v[...]
    lane = lax.iota(jnp.int32, LANES)

    for half in range(N_PASS):
        hbase = half * N_PER_PASS

        @pl.loop(0, G_PER_PASS)
        def _(g):
            iv = iid_v[pl.ds(hbase + g * LANES, LANES)]
            for l in range(LANES):
                j = g * LANES + l
                pltpu.async_copy(q_hbm.at[iv[l]],
                                 rows_v.at[j, pl.ds(N_RANK, N_RANK)], qsem)

        @pl.loop(0, G_PER_PASS)
        def _(g):
            uv = uid_v[pl.ds(hbase + g * LANES, LANES)]
            for sub in range(LANES // CHUNK):
                ss = [uv[sub * CHUNK + l] for l in range(CHUNK)]
                cs = [s // 128 for s in ss]
                fired = [None] * CHUNK
                slots = [None] * CHUNK
                for l in range(CHUNK):
                    if l == 0:
                        fired[l] = None
                        slots[l] = jnp.int32(0)
                    else:
                        f = cs[l] != cs[l - 1]
                        fired[l] = f
                        slots[l] = jnp.where(f, jnp.int32(l), slots[l - 1])
                for l in range(CHUNK):
                    cstart = pl.multiple_of(cs[l] * 128, 128)

                    def fire(l=l, cstart=cstart):
                        for k in range(N_RANK // 8):
                            pltpu.async_copy(
                                pt_hbm.at[k, :, pl.ds(cstart, 128)],
                                blocks_v.at[l, pl.ds(k * 8, 8), :], sem)

                    if fired[l] is None:
                        fire()
                    else:
                        pl.when(fired[l])(fire)
                for l in range(CHUNK):
                    def drain(l=l):
                        for k in range(N_RANK // 8):
                            pltpu.make_async_copy(
                                pt_hbm.at[0, :, pl.ds(0, 128)],
                                blocks_v.at[l, pl.ds(k * 8, 8), :],
                                sem).wait()

                    if fired[l] is None:
                        drain()
                    else:
                        pl.when(fired[l])(drain)
                for l in range(CHUNK):
                    j = g * LANES + sub * CHUNK + l
                    s = ss[l]
                    lane16 = jnp.broadcast_to(s % 128, (LANES,))
                    slot16 = jnp.broadcast_to(slots[l], (LANES,))
                    j16 = jnp.broadcast_to(j, (LANES,))
                    for k in range(N_RANK // LANES):
                        p16 = plsc.load_gather(
                            blocks_v, [slot16, k * LANES + d16, lane16])
                        plsc.store_scatter(rows_v, [j16, k * LANES + d16],
                                           p16)

        @pl.loop(0, N_PER_PASS)
        def _(j):
            pltpu.make_async_copy(q_hbm.at[0],
                                  rows_v.at[j, pl.ds(N_RANK, N_RANK)],
                                  qsem).wait()

        @pl.loop(0, G_PER_PASS)
        def _(g):
            rows = g * LANES + lane
            acc = jnp.zeros((LANES,), jnp.float32)
            for d in range(N_RANK):
                u = plsc.load_gather(
                    rows_v, [rows, jnp.full((LANES,), d, jnp.int32)])
                t = plsc.load_gather(
                    rows_v, [rows, jnp.full((LANES,), N_RANK + d, jnp.int32)])
                x = u * t
                acc = acc + jnp.where(x > 0.0, x,
                                      SELU_ALPHA * (jnp.exp(x) - 1.0))
            sl = pl.ds(hbase + g * LANES, LANES)
            gsl = pl.ds(hbase + g * LANES, LANES)
            out_v[gsl] = SELU_SCALE * acc + ub_v[sl] + ib_v[sl] + mu

    pltpu.async_copy(out_v, out_hbm.at[ord_v], qsem).wait()


def kernel(user_ids, item_ids, P, Q, mu, user_bias, item_bias):
    uid = user_ids.astype(jnp.int32)
    iid = item_ids.astype(jnp.int32)
    order = jnp.argsort(uid).astype(jnp.int32)
    mu16 = jnp.broadcast_to(mu.astype(jnp.float32), (LANES,))
    pt3 = P.T.reshape(8, 8, P.shape[0])
    return _lfm_sc(uid, iid, order, pt3, Q, mu16, user_bias, item_bias)
